# Initial kernel scaffold; baseline (speedup 1.0000x reference)
import jax
import jax.numpy as jnp
from jax.experimental import pallas as pl


def kernel(x, edge_index, W1, att_src1, att_dst1, b1, W2, att_src2, att_dst2, b2):
    raise NotImplementedError("write your pallas kernel here")


# speedup vs baseline: 1.0293x; 1.0293x over previous
import jax
import jax.numpy as jnp
from jax.experimental import pallas as pl

HEADS1 = 8
OUT1 = 16
HEADS2 = 1
OUT2 = 40
N_NODES = 10000
D_IN = 128


def _mm_body(x_ref, w_ref, o_ref):
    o_ref[...] = jnp.dot(x_ref[...], w_ref[...], preferred_element_type=jnp.float32)


def _matmul(x, w):
    n, k = x.shape
    _, m = w.shape
    blk = 400
    return pl.pallas_call(
        _mm_body,
        grid=(n // blk,),
        in_specs=[
            pl.BlockSpec((blk, k), lambda i: (i, 0)),
            pl.BlockSpec((k, m), lambda i: (0, 0)),
        ],
        out_specs=pl.BlockSpec((blk, m), lambda i: (i, 0)),
        out_shape=jax.ShapeDtypeStruct((n, m), jnp.float32),
    )(x, w)


def _gat_layer(x, src, dst, W, att_src, att_dst, bias, heads, out_ch):
    n = x.shape[0]
    h = _matmul(x, W).reshape(n, heads, out_ch)
    a_src = (h * att_src).sum(-1)
    a_dst = (h * att_dst).sum(-1)
    e = a_src[src] + a_dst[dst]
    e = jax.nn.leaky_relu(e, negative_slope=0.2)
    emax = jax.ops.segment_max(e, dst, num_segments=n)
    e = jnp.exp(e - emax[dst])
    denom = jax.ops.segment_sum(e, dst, num_segments=n)
    alpha = e / (denom[dst] + 1e-16)
    msg = h[src] * alpha[..., None]
    out = jax.ops.segment_sum(msg, dst, num_segments=n)
    return out.reshape(n, heads * out_ch) + bias


def kernel(x, edge_index, W1, att_src1, att_dst1, b1, W2, att_src2, att_dst2, b2):
    n = x.shape[0]
    loop = jnp.arange(n, dtype=edge_index.dtype)
    src = jnp.concatenate([edge_index[0], loop])
    dst = jnp.concatenate([edge_index[1], loop])
    h = _gat_layer(x, src, dst, W1, att_src1, att_dst1, b1, HEADS1, OUT1)
    h = jax.nn.relu(h)
    logits = _gat_layer(h, src, dst, W2, att_src2, att_dst2, b2, HEADS2, OUT2)
    return jax.nn.log_softmax(logits, axis=1)


# speedup vs baseline: 35.4377x; 34.4294x over previous
import jax
import jax.numpy as jnp
from jax import lax
from jax.experimental import pallas as pl
from jax.experimental.pallas import tpu as pltpu
from jax.experimental.pallas import tpu_sc as plsc

N = 10000
N_PAD = 10240
E_TOT = 320000 + N
D = 128
H1 = 8
C2 = 40

NC = 2
NS = 16
NW = NC * NS
K = 128
CH = 81
EPW = K * CH
E_PAD = NW * EPW
RPT = N_PAD // NS
N_ACC = 10048
RPA = 632
RPA_LAST = N_ACC - 15 * RPA

_F32 = jnp.float32


def _node1_body(x_ref, w_ref, as_ref, ad_ref, h_ref, s_ref, d_ref):
    h = jnp.dot(x_ref[...], w_ref[...], preferred_element_type=_F32)
    h_ref[...] = h
    s_ref[...] = jnp.dot(h, as_ref[...], preferred_element_type=_F32)
    d_ref[...] = jnp.dot(h, ad_ref[...], preferred_element_type=_F32)


def _node1(xp, W1, As, Ad):
    blk = 512
    return pl.pallas_call(
        _node1_body,
        grid=(N_PAD // blk,),
        in_specs=[
            pl.BlockSpec((blk, D), lambda i: (i, 0)),
            pl.BlockSpec((D, D), lambda i: (0, 0)),
            pl.BlockSpec((D, D), lambda i: (0, 0)),
            pl.BlockSpec((D, D), lambda i: (0, 0)),
        ],
        out_specs=[
            pl.BlockSpec((blk, D), lambda i: (i, 0)),
            pl.BlockSpec((blk, D), lambda i: (i, 0)),
            pl.BlockSpec((blk, D), lambda i: (i, 0)),
        ],
        out_shape=[
            jax.ShapeDtypeStruct((N_PAD, D), _F32),
            jax.ShapeDtypeStruct((N_PAD, D), _F32),
            jax.ShapeDtypeStruct((N_PAD, D), _F32),
        ],
    )(xp, W1, As, Ad)


def _node2_body(m0_ref, m1_ref, d0_ref, d1_ref, e8_ref, b1_ref, w2_ref,
                h2_ref, d_ref):
    m = m0_ref[...] + m1_ref[...]
    d8 = d0_ref[...][:, 0:8] + d1_ref[...][:, 0:8]
    den = jnp.dot(d8, e8_ref[...], preferred_element_type=_F32)
    hr = jnp.maximum(m / (den + 1e-16) + b1_ref[...], 0.0)
    h2 = jnp.dot(hr, w2_ref[...], preferred_element_type=_F32)
    h2_ref[...] = h2
    d_ref[...] = jnp.broadcast_to(h2[:, 41:42], h2.shape)


def _node2(msg, den, E8, b1r, W2e):
    blk = 512
    nb = N_PAD // blk
    return pl.pallas_call(
        _node2_body,
        grid=(nb,),
        in_specs=[
            pl.BlockSpec((blk, D), lambda i: (i, 0)),
            pl.BlockSpec((blk, D), lambda i: (i + nb, 0)),
            pl.BlockSpec((blk, D), lambda i: (i, 0)),
            pl.BlockSpec((blk, D), lambda i: (i + nb, 0)),
            pl.BlockSpec((8, D), lambda i: (0, 0)),
            pl.BlockSpec((1, D), lambda i: (0, 0)),
            pl.BlockSpec((D, D), lambda i: (0, 0)),
        ],
        out_specs=[
            pl.BlockSpec((blk, D), lambda i: (i, 0)),
            pl.BlockSpec((blk, D), lambda i: (i, 0)),
        ],
        out_shape=[
            jax.ShapeDtypeStruct((N_PAD, D), _F32),
            jax.ShapeDtypeStruct((N_PAD, D), _F32),
        ],
    )(msg, msg, den, den, E8, b1r, W2e)


def _final_body(m0_ref, m1_ref, b2_ref, o_ref):
    m = m0_ref[...] + m1_ref[...]
    lg = m[:, 0:40] / (m[:, 40:41] + 1e-16) + b2_ref[...]
    mx = jnp.max(lg, axis=1, keepdims=True)
    s = lg - mx
    o_ref[...] = s - jnp.log(jnp.sum(jnp.exp(s), axis=1, keepdims=True))


def _final(md2, b2r):
    blk = 512
    nb = N_PAD // blk
    return pl.pallas_call(
        _final_body,
        grid=(nb,),
        in_specs=[
            pl.BlockSpec((blk, D), lambda i: (i, 0)),
            pl.BlockSpec((blk, D), lambda i: (i + nb, 0)),
            pl.BlockSpec((1, C2), lambda i: (0, 0)),
        ],
        out_specs=pl.BlockSpec((blk, C2), lambda i: (i, 0)),
        out_shape=jax.ShapeDtypeStruct((N_PAD, C2), _F32),
    )(md2, md2, b2r)


def _l1a_body(src_hbm, dst_hbm, zer_hbm, as_hbm, ad_hbm, den_hbm, w_hbm,
              src_v, dst_v, gs, gd, dbuf, wb_v, accum, s1, s2):
    cid = lax.axis_index("c")
    sid = lax.axis_index("s")
    wid = cid * NS + sid

    @pl.when(sid < NS - 1)
    def _():
        pltpu.sync_copy(zer_hbm.at[pl.ds(0, RPA)],
                        accum.at[pl.ds(sid * RPA, RPA)])

    @pl.when(sid == NS - 1)
    def _():
        pltpu.sync_copy(zer_hbm.at[pl.ds(0, RPA_LAST)],
                        accum.at[pl.ds(15 * RPA, RPA_LAST)])

    pltpu.sync_copy(zer_hbm.at[pl.ds(0, K)], dbuf)
    plsc.subcore_barrier()

    def chunk(g, carry):
        base = wid * EPW + g * K
        pltpu.sync_copy(src_hbm.at[pl.ds(base, K)], src_v)
        pltpu.sync_copy(dst_hbm.at[pl.ds(base, K)], dst_v)
        c1 = pltpu.async_copy(as_hbm.at[src_v], gs, s1)
        c2 = pltpu.async_copy(ad_hbm.at[dst_v], gd, s2)
        c1.wait()
        c2.wait()

        def wrow(k, c):
            v = gs[k, pl.ds(0, 16)] + gd[k, pl.ds(0, 16)]
            v = jnp.maximum(v, 0.2 * v)
            v = jnp.exp(v)
            dbuf[k, pl.ds(0, 16)] = v
            wb_v[pl.ds(k * 8, 16)] = v
            return c

        lax.fori_loop(0, K, wrow, 0)
        pltpu.sync_copy(dbuf, accum.at[dst_v], add=True)
        pltpu.sync_copy(wb_v.at[pl.ds(0, K * 8)], w_hbm.at[pl.ds(base * 8, K * 8)])
        return carry

    lax.fori_loop(0, CH, chunk, 0)
    plsc.subcore_barrier()

    @pl.when(sid < NS - 1)
    def _():
        pltpu.sync_copy(accum.at[pl.ds(sid * RPA, RPA)],
                        den_hbm.at[pl.ds(cid * N_PAD + sid * RPA, RPA)])

    @pl.when(sid == NS - 1)
    def _():
        pltpu.sync_copy(accum.at[pl.ds(15 * RPA, RPA_LAST)],
                        den_hbm.at[pl.ds(cid * N_PAD + 15 * RPA, RPA_LAST)])


def _l1a_call():
    mesh = plsc.VectorSubcoreMesh(core_axis_name="c", subcore_axis_name="s")
    return pl.kernel(
        _l1a_body,
        out_type=[
            jax.ShapeDtypeStruct((NC * N_PAD, D), _F32),
            jax.ShapeDtypeStruct((E_PAD * 8,), _F32),
        ],
        mesh=mesh,
        scratch_types=[
            pltpu.VMEM((K,), jnp.int32),
            pltpu.VMEM((K,), jnp.int32),
            pltpu.VMEM((K, D), _F32),
            pltpu.VMEM((K, D), _F32),
            pltpu.VMEM((K, D), _F32),
            pltpu.VMEM((K * 8 + 16,), _F32),
            pltpu.VMEM_SHARED((N_ACC, D), _F32),
            pltpu.SemaphoreType.DMA,
            pltpu.SemaphoreType.DMA,
        ],
    )


def _l1b_body(src_hbm, dst_hbm, zer_hbm, w_hbm, h_hbm, out_hbm,
              src_v, dst_v, hbuf, mbuf, wb_v, accum, s3):
    cid = lax.axis_index("c")
    sid = lax.axis_index("s")
    wid = cid * NS + sid
    pltpu.sync_copy(zer_hbm, accum.at[pl.ds(sid * RPT, RPT)])
    plsc.subcore_barrier()

    def chunk(g, carry):
        base = wid * EPW + g * K
        pltpu.sync_copy(src_hbm.at[pl.ds(base, K)], src_v)
        pltpu.sync_copy(dst_hbm.at[pl.ds(base, K)], dst_v)
        pltpu.sync_copy(w_hbm.at[pl.ds(base * 8, K * 8)], wb_v.at[pl.ds(0, K * 8)])
        c3 = pltpu.async_copy(h_hbm.at[src_v], hbuf, s3)
        c3.wait()

        def mrow(r, c):
            for jj in range(8):
                wpair = wb_v[pl.ds(r * 128 + jj * 16, 16)]
                for p in range(2):
                    k = r * 16 + jj * 2 + p
                    for h in range(H1):
                        ws = wpair[p * 8 + h]
                        mbuf[k, pl.ds(h * 16, 16)] = (
                            hbuf[k, pl.ds(h * 16, 16)] * ws)
            return c

        lax.fori_loop(0, K // 16, mrow, 0)
        pltpu.sync_copy(mbuf, accum.at[dst_v], add=True)
        return carry

    lax.fori_loop(0, CH, chunk, 0)
    plsc.subcore_barrier()
    row0 = cid * N_PAD + sid * RPT
    pltpu.sync_copy(accum.at[pl.ds(sid * RPT, RPT)],
                    out_hbm.at[pl.ds(row0, RPT)])


def _l1b_call():
    mesh = plsc.VectorSubcoreMesh(core_axis_name="c", subcore_axis_name="s")
    return pl.kernel(
        _l1b_body,
        out_type=jax.ShapeDtypeStruct((NC * N_PAD, D), _F32),
        mesh=mesh,
        scratch_types=[
            pltpu.VMEM((K,), jnp.int32),
            pltpu.VMEM((K,), jnp.int32),
            pltpu.VMEM((K, D), _F32),
            pltpu.VMEM((K, D), _F32),
            pltpu.VMEM((K * 8,), _F32),
            pltpu.VMEM_SHARED((N_PAD, D), _F32),
            pltpu.SemaphoreType.DMA,
        ],
    )


def _l2_body(src_hbm, dst_hbm, zer_hbm, a2d_hbm, h2_hbm, out_hbm,
             src_v, dst_v, gd, hbuf, mbuf, accum, s2, s3):
    cid = lax.axis_index("c")
    sid = lax.axis_index("s")
    wid = cid * NS + sid

    @pl.when(sid < NS - 1)
    def _():
        pltpu.sync_copy(zer_hbm.at[pl.ds(0, RPA)],
                        accum.at[pl.ds(sid * RPA, RPA)])

    @pl.when(sid == NS - 1)
    def _():
        pltpu.sync_copy(zer_hbm.at[pl.ds(0, RPA_LAST)],
                        accum.at[pl.ds(15 * RPA, RPA_LAST)])

    pltpu.sync_copy(zer_hbm.at[pl.ds(0, K)], mbuf)
    plsc.subcore_barrier()
    iot = lax.iota(jnp.int32, 16)

    def chunk(g, carry):
        base = wid * EPW + g * K
        pltpu.sync_copy(src_hbm.at[pl.ds(base, K)], src_v)
        pltpu.sync_copy(dst_hbm.at[pl.ds(base, K)], dst_v)
        c2 = pltpu.async_copy(a2d_hbm.at[dst_v], gd, s2)
        c3 = pltpu.async_copy(h2_hbm.at[src_v], hbuf, s3)
        c2.wait()
        c3.wait()

        def mrow(r, c):
            for j in range(16):
                k = r * 16 + j
                h2v2 = hbuf[k, pl.ds(32, 16)]
                vw = gd[k, pl.ds(0, 16)] + h2v2[8]
                vw = jnp.maximum(vw, 0.2 * vw)
                vw = jnp.exp(vw)
                ws = vw[0]
                mbuf[k, pl.ds(0, 16)] = hbuf[k, pl.ds(0, 16)] * ws
                mbuf[k, pl.ds(16, 16)] = hbuf[k, pl.ds(16, 16)] * ws
                v2 = h2v2 * ws
                v2 = jnp.where(iot == 8, ws, v2)
                mbuf[k, pl.ds(32, 16)] = v2
            return c

        lax.fori_loop(0, K // 16, mrow, 0)
        pltpu.sync_copy(mbuf, accum.at[dst_v], add=True)
        return carry

    lax.fori_loop(0, CH, chunk, 0)
    plsc.subcore_barrier()

    @pl.when(sid < NS - 1)
    def _():
        pltpu.sync_copy(accum.at[pl.ds(sid * RPA, RPA)],
                        out_hbm.at[pl.ds(cid * N_PAD + sid * RPA, RPA)])

    @pl.when(sid == NS - 1)
    def _():
        pltpu.sync_copy(accum.at[pl.ds(15 * RPA, RPA_LAST)],
                        out_hbm.at[pl.ds(cid * N_PAD + 15 * RPA, RPA_LAST)])


def _l2_call():
    mesh = plsc.VectorSubcoreMesh(core_axis_name="c", subcore_axis_name="s")
    return pl.kernel(
        _l2_body,
        out_type=jax.ShapeDtypeStruct((NC * N_PAD, D), _F32),
        mesh=mesh,
        scratch_types=[
            pltpu.VMEM((K,), jnp.int32),
            pltpu.VMEM((K,), jnp.int32),
            pltpu.VMEM((K, D), _F32),
            pltpu.VMEM((K, D), _F32),
            pltpu.VMEM((K, D), _F32),
            pltpu.VMEM_SHARED((N_ACC, D), _F32),
            pltpu.SemaphoreType.DMA,
            pltpu.SemaphoreType.DMA,
        ],
    )


def kernel(x, edge_index, W1, att_src1, att_dst1, b1, W2, att_src2, att_dst2, b2):
    xp = jnp.pad(x.astype(_F32), ((0, N_PAD - N), (0, 0)))
    loop = jnp.arange(N, dtype=jnp.int32)
    pad = jnp.full((E_PAD - E_TOT,), N, jnp.int32)
    src = jnp.concatenate([edge_index[0].astype(jnp.int32), loop, pad])
    dst = jnp.concatenate([edge_index[1].astype(jnp.int32), loop, pad])

    eye8 = jnp.eye(H1, dtype=_F32)
    As = jnp.pad(
        jnp.einsum("hc,hg->hcg", att_src1[0].astype(_F32), eye8).reshape(D, H1),
        ((0, 0), (0, D - H1)))
    Ad = jnp.pad(
        jnp.einsum("hc,hg->hcg", att_dst1[0].astype(_F32), eye8).reshape(D, H1),
        ((0, 0), (0, D - H1)))
    E8 = jnp.repeat(eye8, 16, axis=1)
    vs2 = W2.astype(_F32) @ att_src2[0, 0].astype(_F32)
    vd2 = W2.astype(_F32) @ att_dst2[0, 0].astype(_F32)
    W2e = jnp.concatenate(
        [W2.astype(_F32), vs2[:, None], vd2[:, None],
         jnp.zeros((D, D - C2 - 2), _F32)], axis=1)
    zer = jnp.zeros((RPT, D), _F32)

    h1, ast, adt = _node1(xp, W1.astype(_F32), As, Ad)
    den1, w1e = _l1a_call()(src, dst, zer, ast, adt)
    msg1 = _l1b_call()(src, dst, zer, w1e, h1)
    h2e, a2d = _node2(msg1, den1, E8, b1.astype(_F32).reshape(1, D), W2e)
    md2 = _l2_call()(src, dst, zer, a2d, h2e)
    out = _final(md2, b2.astype(_F32).reshape(1, C2))
    return out[:N]


# speedup vs baseline: 35.9918x; 1.0156x over previous
import jax
import jax.numpy as jnp
from jax import lax
from jax.experimental import pallas as pl
from jax.experimental.pallas import tpu as pltpu
from jax.experimental.pallas import tpu_sc as plsc

N = 10000
N_PAD = 10240
E_TOT = 320000 + N
D = 128
H1 = 8
C2 = 40

NC = 2
NS = 16
NW = NC * NS
K = 64
CH = 163
EPW = K * CH
E_PAD = NW * EPW
RPT = N_PAD // NS
N_ACC = 10008
RPA = 632
RPA_LAST = N_ACC - 15 * RPA

_F32 = jnp.float32


def _node1_body(x_ref, w_ref, as_ref, ad_ref, h_ref, s_ref, d_ref):
    h = jnp.dot(x_ref[...], w_ref[...], preferred_element_type=_F32)
    h_ref[...] = h
    s_ref[...] = jnp.dot(h, as_ref[...], preferred_element_type=_F32)
    d_ref[...] = jnp.dot(h, ad_ref[...], preferred_element_type=_F32)


def _node1(xp, W1, As, Ad):
    blk = 512
    return pl.pallas_call(
        _node1_body,
        grid=(N_PAD // blk,),
        in_specs=[
            pl.BlockSpec((blk, D), lambda i: (i, 0)),
            pl.BlockSpec((D, D), lambda i: (0, 0)),
            pl.BlockSpec((D, D), lambda i: (0, 0)),
            pl.BlockSpec((D, D), lambda i: (0, 0)),
        ],
        out_specs=[
            pl.BlockSpec((blk, D), lambda i: (i, 0)),
            pl.BlockSpec((blk, D), lambda i: (i, 0)),
            pl.BlockSpec((blk, D), lambda i: (i, 0)),
        ],
        out_shape=[
            jax.ShapeDtypeStruct((N_PAD, D), _F32),
            jax.ShapeDtypeStruct((N_PAD, D), _F32),
            jax.ShapeDtypeStruct((N_PAD, D), _F32),
        ],
    )(xp, W1, As, Ad)


def _node2_body(m0_ref, m1_ref, d0_ref, d1_ref, e8_ref, b1_ref, w2_ref,
                h2_ref, d_ref):
    m = m0_ref[...] + m1_ref[...]
    d8 = d0_ref[...][:, 0:8] + d1_ref[...][:, 0:8]
    den = jnp.dot(d8, e8_ref[...], preferred_element_type=_F32)
    hr = jnp.maximum(m / (den + 1e-16) + b1_ref[...], 0.0)
    h2 = jnp.dot(hr, w2_ref[...], preferred_element_type=_F32)
    h2_ref[...] = h2
    d_ref[...] = jnp.broadcast_to(h2[:, 41:42], h2.shape)


def _node2(msg, den, E8, b1r, W2e):
    blk = 512
    nb = N_PAD // blk
    return pl.pallas_call(
        _node2_body,
        grid=(nb,),
        in_specs=[
            pl.BlockSpec((blk, D), lambda i: (i, 0)),
            pl.BlockSpec((blk, D), lambda i: (i + nb, 0)),
            pl.BlockSpec((blk, D), lambda i: (i, 0)),
            pl.BlockSpec((blk, D), lambda i: (i + nb, 0)),
            pl.BlockSpec((8, D), lambda i: (0, 0)),
            pl.BlockSpec((1, D), lambda i: (0, 0)),
            pl.BlockSpec((D, D), lambda i: (0, 0)),
        ],
        out_specs=[
            pl.BlockSpec((blk, D), lambda i: (i, 0)),
            pl.BlockSpec((blk, D), lambda i: (i, 0)),
        ],
        out_shape=[
            jax.ShapeDtypeStruct((N_PAD, D), _F32),
            jax.ShapeDtypeStruct((N_PAD, D), _F32),
        ],
    )(msg, msg, den, den, E8, b1r, W2e)


def _final_body(m0_ref, m1_ref, b2_ref, o_ref):
    m = m0_ref[...] + m1_ref[...]
    lg = m[:, 0:40] / (m[:, 40:41] + 1e-16) + b2_ref[...]
    mx = jnp.max(lg, axis=1, keepdims=True)
    s = lg - mx
    o_ref[...] = s - jnp.log(jnp.sum(jnp.exp(s), axis=1, keepdims=True))


def _final(md2, b2r):
    blk = 512
    nb = N_PAD // blk
    return pl.pallas_call(
        _final_body,
        grid=(nb,),
        in_specs=[
            pl.BlockSpec((blk, D), lambda i: (i, 0)),
            pl.BlockSpec((blk, D), lambda i: (i + nb, 0)),
            pl.BlockSpec((1, C2), lambda i: (0, 0)),
        ],
        out_specs=pl.BlockSpec((blk, C2), lambda i: (i, 0)),
        out_shape=jax.ShapeDtypeStruct((N_PAD, C2), _F32),
    )(md2, md2, b2r)


def _l1a_body(src_hbm, dst_hbm, zer_hbm, as_hbm, ad_hbm, den_hbm, w_hbm,
              src0, dst0, gs0, gd0, db0, wb0,
              src1, dst1, gs1, gd1, db1, wb1, accum, s0, s1):
    cid = lax.axis_index("c")
    sid = lax.axis_index("s")
    wid = cid * NS + sid

    @pl.when(sid < NS - 1)
    def _():
        pltpu.sync_copy(zer_hbm.at[pl.ds(0, RPA)],
                        accum.at[pl.ds(sid * RPA, RPA)])

    @pl.when(sid == NS - 1)
    def _():
        pltpu.sync_copy(zer_hbm.at[pl.ds(0, RPA_LAST)],
                        accum.at[pl.ds(15 * RPA, RPA_LAST)])

    pltpu.sync_copy(zer_hbm.at[pl.ds(0, K)], db0)
    pltpu.sync_copy(zer_hbm.at[pl.ds(0, K)], db1)
    plsc.subcore_barrier()

    B0 = (src0, dst0, gs0, gd0, db0, wb0, s0)
    B1 = (src1, dst1, gs1, gd1, db1, wb1, s1)

    def start(bs, g):
        src_v, dst_v, gs, gd, _, _, sem = bs
        base = wid * EPW + g * K
        pltpu.sync_copy(src_hbm.at[pl.ds(base, K)], src_v)
        pltpu.sync_copy(dst_hbm.at[pl.ds(base, K)], dst_v)
        pltpu.async_copy(as_hbm.at[src_v], gs, sem)
        pltpu.async_copy(ad_hbm.at[dst_v], gd, sem)

    def wait(bs):
        src_v, dst_v, gs, gd, _, _, sem = bs
        pltpu.make_async_copy(as_hbm.at[src_v], gs, sem).wait()
        pltpu.make_async_copy(ad_hbm.at[dst_v], gd, sem).wait()

    def proc(bs, g):
        src_v, dst_v, gs, gd, dbuf, wb_v, _ = bs
        base = wid * EPW + g * K

        def wrow(r, c):
            for j in range(16):
                k = r * 16 + j
                v = gs[k, pl.ds(0, 16)] + gd[k, pl.ds(0, 16)]
                v = jnp.maximum(v, 0.2 * v)
                v = jnp.exp(v)
                dbuf[k, pl.ds(0, 16)] = v
                wb_v[pl.ds(k * 8, 16)] = v
            return c

        lax.fori_loop(0, K // 16, wrow, 0)
        pltpu.sync_copy(dbuf, accum.at[dst_v], add=True)
        pltpu.sync_copy(wb_v.at[pl.ds(0, K * 8)],
                        w_hbm.at[pl.ds(base * 8, K * 8)])

    start(B0, 0)

    def pair(i, carry):
        g = 2 * i
        start(B1, g + 1)
        wait(B0)
        proc(B0, g)
        start(B0, g + 2)
        wait(B1)
        proc(B1, g + 1)
        return carry

    lax.fori_loop(0, (CH - 1) // 2, pair, 0)
    wait(B0)
    proc(B0, CH - 1)
    plsc.subcore_barrier()

    @pl.when(sid < NS - 1)
    def _():
        pltpu.sync_copy(accum.at[pl.ds(sid * RPA, RPA)],
                        den_hbm.at[pl.ds(cid * N_PAD + sid * RPA, RPA)])

    @pl.when(sid == NS - 1)
    def _():
        pltpu.sync_copy(accum.at[pl.ds(15 * RPA, RPA_LAST)],
                        den_hbm.at[pl.ds(cid * N_PAD + 15 * RPA, RPA_LAST)])


def _l1a_call():
    mesh = plsc.VectorSubcoreMesh(core_axis_name="c", subcore_axis_name="s")
    buf = [
        pltpu.VMEM((K,), jnp.int32),
        pltpu.VMEM((K,), jnp.int32),
        pltpu.VMEM((K, D), _F32),
        pltpu.VMEM((K, D), _F32),
        pltpu.VMEM((K, D), _F32),
        pltpu.VMEM((K * 8 + 16,), _F32),
    ]
    return pl.kernel(
        _l1a_body,
        out_type=[
            jax.ShapeDtypeStruct((NC * N_PAD, D), _F32),
            jax.ShapeDtypeStruct((E_PAD * 8,), _F32),
        ],
        mesh=mesh,
        scratch_types=buf + buf + [
            pltpu.VMEM_SHARED((N_ACC, D), _F32),
            pltpu.SemaphoreType.DMA,
            pltpu.SemaphoreType.DMA,
        ],
    )


def _l1b_body(src_hbm, dst_hbm, zer_hbm, w_hbm, h_hbm, out_hbm,
              src0, dst0, hb0, mb0, wb0,
              src1, dst1, hb1, mb1, wb1, accum, s0, s1):
    cid = lax.axis_index("c")
    sid = lax.axis_index("s")
    wid = cid * NS + sid
    pltpu.sync_copy(zer_hbm, accum.at[pl.ds(sid * RPT, RPT)])
    plsc.subcore_barrier()

    B0 = (src0, dst0, hb0, mb0, wb0, s0)
    B1 = (src1, dst1, hb1, mb1, wb1, s1)

    def start(bs, g):
        src_v, dst_v, hbuf, _, wb_v, sem = bs
        base = wid * EPW + g * K
        pltpu.sync_copy(src_hbm.at[pl.ds(base, K)], src_v)
        pltpu.sync_copy(dst_hbm.at[pl.ds(base, K)], dst_v)
        pltpu.sync_copy(w_hbm.at[pl.ds(base * 8, K * 8)],
                        wb_v.at[pl.ds(0, K * 8)])
        pltpu.async_copy(h_hbm.at[src_v], hbuf, sem)

    def wait(bs):
        src_v, _, hbuf, _, _, sem = bs
        pltpu.make_async_copy(h_hbm.at[src_v], hbuf, sem).wait()

    def proc(bs):
        _, dst_v, hbuf, mbuf, wb_v, _ = bs

        def mrow(r, c):
            for jj in range(8):
                wpair = wb_v[pl.ds(r * 128 + jj * 16, 16)]
                for p in range(2):
                    k = r * 16 + jj * 2 + p
                    for h in range(H1):
                        ws = wpair[p * 8 + h]
                        mbuf[k, pl.ds(h * 16, 16)] = (
                            hbuf[k, pl.ds(h * 16, 16)] * ws)
            return c

        lax.fori_loop(0, K // 16, mrow, 0)
        pltpu.sync_copy(mbuf, accum.at[dst_v], add=True)

    start(B0, 0)

    def pair(i, carry):
        g = 2 * i
        start(B1, g + 1)
        wait(B0)
        proc(B0)
        start(B0, g + 2)
        wait(B1)
        proc(B1)
        return carry

    lax.fori_loop(0, (CH - 1) // 2, pair, 0)
    wait(B0)
    proc(B0)
    plsc.subcore_barrier()
    row0 = cid * N_PAD + sid * RPT
    pltpu.sync_copy(accum.at[pl.ds(sid * RPT, RPT)],
                    out_hbm.at[pl.ds(row0, RPT)])


def _l1b_call():
    mesh = plsc.VectorSubcoreMesh(core_axis_name="c", subcore_axis_name="s")
    buf = [
        pltpu.VMEM((K,), jnp.int32),
        pltpu.VMEM((K,), jnp.int32),
        pltpu.VMEM((K, D), _F32),
        pltpu.VMEM((K, D), _F32),
        pltpu.VMEM((K * 8,), _F32),
    ]
    return pl.kernel(
        _l1b_body,
        out_type=jax.ShapeDtypeStruct((NC * N_PAD, D), _F32),
        mesh=mesh,
        scratch_types=buf + buf + [
            pltpu.VMEM_SHARED((N_PAD, D), _F32),
            pltpu.SemaphoreType.DMA,
            pltpu.SemaphoreType.DMA,
        ],
    )


def _l2_body(src_hbm, dst_hbm, zer_hbm, a2d_hbm, h2_hbm, out_hbm,
             src0, dst0, gd0, hb0, mb0,
             src1, dst1, gd1, hb1, mb1, accum, s0, s1):
    cid = lax.axis_index("c")
    sid = lax.axis_index("s")
    wid = cid * NS + sid

    @pl.when(sid < NS - 1)
    def _():
        pltpu.sync_copy(zer_hbm.at[pl.ds(0, RPA)],
                        accum.at[pl.ds(sid * RPA, RPA)])

    @pl.when(sid == NS - 1)
    def _():
        pltpu.sync_copy(zer_hbm.at[pl.ds(0, RPA_LAST)],
                        accum.at[pl.ds(15 * RPA, RPA_LAST)])

    pltpu.sync_copy(zer_hbm.at[pl.ds(0, K)], mb0)
    pltpu.sync_copy(zer_hbm.at[pl.ds(0, K)], mb1)
    plsc.subcore_barrier()
    iot = lax.iota(jnp.int32, 16)

    B0 = (src0, dst0, gd0, hb0, mb0, s0)
    B1 = (src1, dst1, gd1, hb1, mb1, s1)

    def start(bs, g):
        src_v, dst_v, gd, hbuf, _, sem = bs
        base = wid * EPW + g * K
        pltpu.sync_copy(src_hbm.at[pl.ds(base, K)], src_v)
        pltpu.sync_copy(dst_hbm.at[pl.ds(base, K)], dst_v)
        pltpu.async_copy(a2d_hbm.at[dst_v], gd, sem)
        pltpu.async_copy(h2_hbm.at[src_v], hbuf, sem)

    def wait(bs):
        src_v, dst_v, gd, hbuf, _, sem = bs
        pltpu.make_async_copy(a2d_hbm.at[dst_v], gd, sem).wait()
        pltpu.make_async_copy(h2_hbm.at[src_v], hbuf, sem).wait()

    def proc(bs):
        _, dst_v, gd, hbuf, mbuf, _ = bs

        def mrow(r, c):
            for j in range(16):
                k = r * 16 + j
                h2v2 = hbuf[k, pl.ds(32, 16)]
                vw = gd[k, pl.ds(0, 16)] + h2v2[8]
                vw = jnp.maximum(vw, 0.2 * vw)
                vw = jnp.exp(vw)
                ws = vw[0]
                mbuf[k, pl.ds(0, 16)] = hbuf[k, pl.ds(0, 16)] * ws
                mbuf[k, pl.ds(16, 16)] = hbuf[k, pl.ds(16, 16)] * ws
                v2 = h2v2 * ws
                v2 = jnp.where(iot == 8, ws, v2)
                mbuf[k, pl.ds(32, 16)] = v2
            return c

        lax.fori_loop(0, K // 16, mrow, 0)
        pltpu.sync_copy(mbuf, accum.at[dst_v], add=True)

    start(B0, 0)

    def pair(i, carry):
        g = 2 * i
        start(B1, g + 1)
        wait(B0)
        proc(B0)
        start(B0, g + 2)
        wait(B1)
        proc(B1)
        return carry

    lax.fori_loop(0, (CH - 1) // 2, pair, 0)
    wait(B0)
    proc(B0)
    plsc.subcore_barrier()

    @pl.when(sid < NS - 1)
    def _():
        pltpu.sync_copy(accum.at[pl.ds(sid * RPA, RPA)],
                        out_hbm.at[pl.ds(cid * N_PAD + sid * RPA, RPA)])

    @pl.when(sid == NS - 1)
    def _():
        pltpu.sync_copy(accum.at[pl.ds(15 * RPA, RPA_LAST)],
                        out_hbm.at[pl.ds(cid * N_PAD + 15 * RPA, RPA_LAST)])


def _l2_call():
    mesh = plsc.VectorSubcoreMesh(core_axis_name="c", subcore_axis_name="s")
    buf = [
        pltpu.VMEM((K,), jnp.int32),
        pltpu.VMEM((K,), jnp.int32),
        pltpu.VMEM((K, D), _F32),
        pltpu.VMEM((K, D), _F32),
        pltpu.VMEM((K, D), _F32),
    ]
    return pl.kernel(
        _l2_body,
        out_type=jax.ShapeDtypeStruct((NC * N_PAD, D), _F32),
        mesh=mesh,
        scratch_types=buf + buf + [
            pltpu.VMEM_SHARED((N_ACC, D), _F32),
            pltpu.SemaphoreType.DMA,
            pltpu.SemaphoreType.DMA,
        ],
    )


def kernel(x, edge_index, W1, att_src1, att_dst1, b1, W2, att_src2, att_dst2, b2):
    xp = jnp.pad(x.astype(_F32), ((0, N_PAD - N), (0, 0)))
    loop = jnp.arange(N, dtype=jnp.int32)
    pad = jnp.full((E_PAD - E_TOT,), N, jnp.int32)
    src = jnp.concatenate([edge_index[0].astype(jnp.int32), loop, pad])
    dst = jnp.concatenate([edge_index[1].astype(jnp.int32), loop, pad])

    eye8 = jnp.eye(H1, dtype=_F32)
    As = jnp.pad(
        jnp.einsum("hc,hg->hcg", att_src1[0].astype(_F32), eye8).reshape(D, H1),
        ((0, 0), (0, D - H1)))
    Ad = jnp.pad(
        jnp.einsum("hc,hg->hcg", att_dst1[0].astype(_F32), eye8).reshape(D, H1),
        ((0, 0), (0, D - H1)))
    E8 = jnp.repeat(eye8, 16, axis=1)
    vs2 = W2.astype(_F32) @ att_src2[0, 0].astype(_F32)
    vd2 = W2.astype(_F32) @ att_dst2[0, 0].astype(_F32)
    W2e = jnp.concatenate(
        [W2.astype(_F32), vs2[:, None], vd2[:, None],
         jnp.zeros((D, D - C2 - 2), _F32)], axis=1)
    zer = jnp.zeros((RPT, D), _F32)

    h1, ast, adt = _node1(xp, W1.astype(_F32), As, Ad)
    den1, w1e = _l1a_call()(src, dst, zer, ast, adt)
    msg1 = _l1b_call()(src, dst, zer, w1e, h1)
    h2e, a2d = _node2(msg1, den1, E8, b1.astype(_F32).reshape(1, D), W2e)
    md2 = _l2_call()(src, dst, zer, a2d, h2e)
    out = _final(md2, b2.astype(_F32).reshape(1, C2))
    return out[:N]
